# baseline (speedup 1.0000x reference)
import functools

import jax
import jax.numpy as jnp
from jax import lax
from jax.experimental import pallas as pl
from jax.experimental.pallas import tpu as pltpu
from jax.experimental.pallas import tpu_sc as plsc

B = 4096
S = 50
P = 20
T = P + S
D = 128

_info = plsc.get_sparse_core_info()
NC, NS = _info.num_cores, _info.num_subcores
NW = NC * NS
ROWS_PER_W = B // NW
CHUNK = 4
NBUF = 2
NSTEPS = ROWS_PER_W // CHUNK
NOUT = NSTEPS // NBUF


def _make_kernel():
    mesh = plsc.VectorSubcoreMesh(core_axis_name="c", subcore_axis_name="s")

    @functools.partial(
        pl.kernel,
        mesh=mesh,
        compiler_params=pltpu.CompilerParams(use_tc_tiling_on_sc=True),
        out_type=jax.ShapeDtypeStruct((B * T, D), jnp.float32),
        scratch_types=[
            pltpu.VMEM((ROWS_PER_W, S), jnp.int32),
            pltpu.VMEM((CHUNK * T, D), jnp.float32),
            pltpu.VMEM((CHUNK * T, D), jnp.float32),
            pltpu.SemaphoreType.DMA,
            pltpu.SemaphoreType.DMA,
            pltpu.SemaphoreType.DMA,
            pltpu.SemaphoreType.DMA,
        ],
    )
    def k(ids_hbm, table_hbm, prompt_hbm, out_hbm,
          idx_v, buf0, buf1, g0, g1, w0, w1):
        bufs = (buf0, buf1)
        gsems = (g0, g1)
        wsems = (w0, w1)
        wid = lax.axis_index("s") * NC + lax.axis_index("c")
        base = wid * ROWS_PER_W

        pltpu.sync_copy(ids_hbm.at[pl.ds(base, ROWS_PER_W)], idx_v)

        for s in range(NBUF):
            for r in range(CHUNK):
                pltpu.sync_copy(prompt_hbm, bufs[s].at[pl.ds(r * T, P)])

        def fire_gathers(cc, s):
            for r in range(CHUNK):
                pltpu.async_copy(
                    table_hbm.at[idx_v.at[cc * CHUNK + r]],
                    bufs[s].at[pl.ds(r * T + P, S)],
                    gsems[s],
                )

        def drain_gathers(s):
            pltpu.make_async_copy(
                table_hbm.at[pl.ds(0, CHUNK * S)],
                bufs[s].at[pl.ds(0, CHUNK * S)],
                gsems[s],
            ).wait()

        def fire_write(cc, s):
            pltpu.async_copy(
                bufs[s],
                out_hbm.at[pl.ds((base + cc * CHUNK) * T, CHUNK * T)],
                wsems[s],
            )

        def drain_write(s):
            pltpu.make_async_copy(
                out_hbm.at[pl.ds(0, CHUNK * T)],
                bufs[s],
                wsems[s],
            ).wait()

        for s in range(NBUF):
            fire_gathers(s, s)

        def outer(c, carry):
            cc0 = c * NBUF
            for s in range(NBUF):
                drain_gathers(s)
                fire_write(cc0 + s, s)
            for s in range(NBUF):
                drain_write(s)
                fire_gathers(cc0 + NBUF + s, s)
            return carry

        lax.fori_loop(0, NOUT - 1, outer, 0)

        for s in range(NBUF):
            drain_gathers(s)
            fire_write(NSTEPS - NBUF + s, s)
        for s in range(NBUF):
            drain_write(s)

    return k


_kernel = _make_kernel()


def kernel(input_ids, embed_table, prompt_weight):
    ids = input_ids.astype(jnp.int32)
    out = _kernel(ids, embed_table, prompt_weight)
    return out.reshape(B, T, D)


# speedup vs baseline: 1.7509x; 1.7509x over previous
import functools

import jax
import jax.numpy as jnp
from jax import lax
from jax.experimental import pallas as pl
from jax.experimental.pallas import tpu as pltpu
from jax.experimental.pallas import tpu_sc as plsc

B = 4096
S = 50
P = 20
T = P + S
D = 128

_info = plsc.get_sparse_core_info()
NC, NS = _info.num_cores, _info.num_subcores
NW = NC * NS
ROWS_PER_W = B // NW
CHUNK = 4
NBUF = 2
NSTEPS = ROWS_PER_W // CHUNK
NOUT = NSTEPS // NBUF


def _make_kernel():
    mesh = plsc.VectorSubcoreMesh(core_axis_name="c", subcore_axis_name="s")

    @functools.partial(
        pl.kernel,
        mesh=mesh,
        compiler_params=pltpu.CompilerParams(use_tc_tiling_on_sc=True),
        out_type=jax.ShapeDtypeStruct((B, T, D), jnp.float32),
        scratch_types=[
            pltpu.VMEM((ROWS_PER_W, S), jnp.int32),
            pltpu.VMEM((CHUNK * T, D), jnp.float32),
            pltpu.VMEM((CHUNK * T, D), jnp.float32),
            pltpu.SemaphoreType.DMA,
            pltpu.SemaphoreType.DMA,
            pltpu.SemaphoreType.DMA,
            pltpu.SemaphoreType.DMA,
        ],
    )
    def k(ids_hbm, table_hbm, prompt_hbm, out_hbm,
          idx_v, buf0, buf1, g0, g1, w0, w1):
        bufs = (buf0, buf1)
        gsems = (g0, g1)
        wsems = (w0, w1)
        wid = lax.axis_index("s") * NC + lax.axis_index("c")
        base = wid * ROWS_PER_W

        pltpu.sync_copy(ids_hbm.at[pl.ds(base, ROWS_PER_W)], idx_v)

        for s in range(NBUF):
            for r in range(CHUNK):
                pltpu.sync_copy(prompt_hbm, bufs[s].at[pl.ds(r * T, P)])

        def fire_gathers(cc, s):
            for r in range(CHUNK):
                pltpu.async_copy(
                    table_hbm.at[idx_v.at[cc * CHUNK + r]],
                    bufs[s].at[pl.ds(r * T + P, S)],
                    gsems[s],
                )

        def drain_gathers(s):
            pltpu.make_async_copy(
                table_hbm.at[pl.ds(0, CHUNK * S)],
                bufs[s].at[pl.ds(0, CHUNK * S)],
                gsems[s],
            ).wait()

        def fire_write(cc, s):
            for r in range(CHUNK):
                pltpu.async_copy(
                    bufs[s].at[pl.ds(r * T, T)],
                    out_hbm.at[base + cc * CHUNK + r],
                    wsems[s],
                )

        def drain_write(s):
            for r in range(CHUNK):
                pltpu.make_async_copy(
                    bufs[s].at[pl.ds(r * T, T)],
                    out_hbm.at[r],
                    wsems[s],
                ).wait()

        for s in range(NBUF):
            fire_gathers(s, s)

        def outer(c, carry):
            cc0 = c * NBUF
            for s in range(NBUF):
                drain_gathers(s)
                fire_write(cc0 + s, s)
            for s in range(NBUF):
                drain_write(s)
                fire_gathers(cc0 + NBUF + s, s)
            return carry

        lax.fori_loop(0, NOUT - 1, outer, 0)

        for s in range(NBUF):
            drain_gathers(s)
            fire_write(NSTEPS - NBUF + s, s)
        for s in range(NBUF):
            drain_write(s)

    return k


_kernel = _make_kernel()


def kernel(input_ids, embed_table, prompt_weight):
    ids = input_ids.astype(jnp.int32)
    return _kernel(ids, embed_table, prompt_weight)
